# Initial kernel scaffold; baseline (speedup 1.0000x reference)
import jax
import jax.numpy as jnp
from jax.experimental import pallas as pl


def kernel(x, emb, lin_w, lin_b, W1, b1, W2, b2):
    raise NotImplementedError("write your pallas kernel here")


# speedup vs baseline: 6.9443x; 6.9443x over previous
import functools

import jax
import jax.numpy as jnp
from jax import lax
from jax.experimental import pallas as pl
from jax.experimental.pallas import tpu as pltpu
from jax.experimental.pallas import tpu_sc as plsc

B = 16384
F = 26
V = 100000
D = 16
H = 128
BF = B * F
EMBED_OUT = F * D

NC = 2
NS = 16
NW = NC * NS

RPW = B // NW
IPW = RPW * F
GCH = 128
NG = IPW // GCH
GROUPS = IPW // 16
OFF_PERIOD_GROUPS = 13
OFF_PERIOD = OFF_PERIOD_GROUPS * 16


def _sc_gather(x_hbm, offs_hbm, emb_hbm, lin_hbm, out_hbm, linout_hbm,
               idx_v, offs_v, row_v, lin_v, sem_e, sem_l):
    wid = lax.axis_index("s") * NC + lax.axis_index("c")
    base_i = wid * IPW

    pltpu.sync_copy(x_hbm.at[pl.ds(base_i, IPW)], idx_v)
    pltpu.sync_copy(offs_hbm, offs_v)

    def add_body(i, _):
        for j in range(OFF_PERIOD_GROUPS):
            s = pl.ds(i * OFF_PERIOD + j * 16, 16)
            idx_v[s] = idx_v[s] + offs_v[pl.ds(j * 16, 16)]
        return 0

    lax.fori_loop(0, GROUPS // OFF_PERIOD_GROUPS, add_body, 0)

    def gather_body(g, _):
        isl = pl.ds(g * GCH, GCH)
        ce = pltpu.async_copy(emb_hbm.at[idx_v.at[isl]], row_v, sem_e)
        cl = pltpu.async_copy(lin_hbm.at[idx_v.at[isl]], lin_v.at[isl], sem_l)
        ce.wait()
        pltpu.sync_copy(row_v, out_hbm.at[pl.ds(base_i + g * GCH, GCH)])
        cl.wait()
        return 0

    lax.fori_loop(0, NG, gather_body, 0)

    pltpu.sync_copy(lin_v, linout_hbm.at[pl.ds(base_i, IPW)])


def _sc_call(x_flat, offs, emb, lin_flat):
    mesh = plsc.VectorSubcoreMesh(core_axis_name="c", subcore_axis_name="s",
                                  num_cores=NC, num_subcores=NS)
    return pl.kernel(
        _sc_gather,
        out_type=(jax.ShapeDtypeStruct((BF, D), jnp.float32),
                  jax.ShapeDtypeStruct((BF,), jnp.float32)),
        mesh=mesh,
        scratch_types=[
            pltpu.VMEM((IPW,), jnp.int32),
            pltpu.VMEM((OFF_PERIOD,), jnp.int32),
            pltpu.VMEM((GCH, D), jnp.float32),
            pltpu.VMEM((IPW,), jnp.float32),
            pltpu.SemaphoreType.DMA,
            pltpu.SemaphoreType.DMA,
        ],
        compiler_params=pltpu.CompilerParams(use_tc_tiling_on_sc=False),
    )(x_flat, offs, emb, lin_flat)


BB = 2048


def _mlp_body(flat_ref, lin_ref, w1_ref, b1_ref, w2t_ref, bias_ref, out_ref):
    h = jnp.dot(flat_ref[...], w1_ref[...], preferred_element_type=jnp.float32)
    h = jnp.maximum(h + b1_ref[...], 0.0)
    deep = jnp.sum(h * w2t_ref[...], axis=1, keepdims=True)
    wide = jnp.sum(lin_ref[...], axis=1, keepdims=True)
    out_ref[...] = jax.nn.sigmoid(deep + wide + bias_ref[...])


def _mlp_call(flat, linmat, W1, b1r, W2t, bias):
    grid = (B // BB,)
    return pl.pallas_call(
        _mlp_body,
        grid=grid,
        in_specs=[
            pl.BlockSpec((BB, EMBED_OUT), lambda i: (i, 0)),
            pl.BlockSpec((BB, F), lambda i: (i, 0)),
            pl.BlockSpec((EMBED_OUT, H), lambda i: (0, 0)),
            pl.BlockSpec((1, H), lambda i: (0, 0)),
            pl.BlockSpec((1, H), lambda i: (0, 0)),
            pl.BlockSpec((1, 1), lambda i: (0, 0)),
        ],
        out_specs=pl.BlockSpec((BB, 1), lambda i: (i, 0)),
        out_shape=jax.ShapeDtypeStruct((B, 1), jnp.float32),
    )(flat, linmat, W1, b1r, W2t, bias)


def kernel(x, emb, lin_w, lin_b, W1, b1, W2, b2):
    x_flat = x.astype(jnp.int32).reshape(BF)
    offs = ((jnp.arange(OFF_PERIOD, dtype=jnp.int32) % F) * V)
    lin_flat = lin_w.reshape(-1)
    gathered, lin_gath = _sc_call(x_flat, offs, emb, lin_flat)
    flat = gathered.reshape(B, EMBED_OUT)
    linmat = lin_gath.reshape(B, F)
    bias = (b2 + lin_b).reshape(1, 1)
    out = _mlp_call(flat, linmat, W1, b1.reshape(1, H), W2.reshape(1, H), bias)
    return out.reshape(B)
